# Initial kernel scaffold; baseline (speedup 1.0000x reference)
import jax
import jax.numpy as jnp
from jax.experimental import pallas as pl


def kernel(x, edge_index, edge_attr, W1, b1, gamma, beta, W2, b2):
    raise NotImplementedError("write your pallas kernel here")


# speedup vs baseline: 20.2476x; 20.2476x over previous
import functools
import math

import jax
import jax.numpy as jnp
from jax import lax
from jax.experimental import pallas as pl
from jax.experimental.pallas import tpu as pltpu
from jax.experimental.pallas import tpu_sc as plsc

N = 10000
NPAD = 10240
E = 320000
NC, NS = 2, 16
NW = NC * NS
CH = 128
NCHUNK = 80
EPAD = NW * NCHUNK * CH
D1 = 32
D2 = 48
DW = 128
ROWS_PT = NPAD // NS
CBN = 1.0 / math.sqrt(1.0 + 1e-5)
BLK = 1024

_MESH = plsc.VectorSubcoreMesh(
    core_axis_name="c", subcore_axis_name="s", num_cores=NC, num_subcores=NS
)


def _deg_body(dsts, ews, out, dst_v, ew_v, deg_sh):
    cid = lax.axis_index("c")
    sid = lax.axis_index("s")
    w = sid * NC + cid

    def _z(i, c):
        ew_v[0, pl.ds(i * 16, 16)] = jnp.zeros((16,), jnp.float32)
        return c

    lax.fori_loop(0, CH // 16, _z, 0)
    for t in range(ROWS_PT // CH):
        pltpu.sync_copy(ew_v.at[0], deg_sh.at[pl.ds(sid * ROWS_PT + t * CH, CH)])
    plsc.subcore_barrier()

    pltpu.sync_copy(dsts.at[w], dst_v)
    pltpu.sync_copy(ews.at[w], ew_v)

    def _chunk(j, c):
        pltpu.sync_copy(ew_v.at[j], deg_sh.at[dst_v.at[j]], add=True)
        return c

    lax.fori_loop(0, NCHUNK, _chunk, 0)
    plsc.subcore_barrier()
    pltpu.sync_copy(
        deg_sh.at[pl.ds(sid * ROWS_PT, ROWS_PT)],
        out.at[cid, pl.ds(sid * ROWS_PT, ROWS_PT)],
    )


_deg_call = pl.kernel(
    _deg_body,
    out_type=jax.ShapeDtypeStruct((NC, NPAD), jnp.float32),
    mesh=_MESH,
    scratch_types=[
        pltpu.VMEM((NCHUNK, CH), jnp.int32),
        pltpu.VMEM((NCHUNK, CH), jnp.float32),
        pltpu.VMEM_SHARED((NPAD,), jnp.float32),
    ],
)


def _agg_body(v_hbm, srcs, dsts, ews, out, src_v, dst_v, ew_v, rows, acc_sh,
              sem, *, d):
    cid = lax.axis_index("c")
    sid = lax.axis_index("s")
    w = sid * NC + cid
    nv = d // 16

    def _z(i, c):
        for k in range(DW // 16):
            rows[i, pl.ds(k * 16, 16)] = jnp.zeros((16,), jnp.float32)
        return c

    lax.fori_loop(0, CH, _z, 0)
    for t in range(ROWS_PT // CH):
        pltpu.sync_copy(rows, acc_sh.at[pl.ds(sid * ROWS_PT + t * CH, CH)])
    plsc.subcore_barrier()

    pltpu.sync_copy(srcs.at[w], src_v)
    pltpu.sync_copy(dsts.at[w], dst_v)
    pltpu.sync_copy(ews.at[w], ew_v)

    def _chunk(j, c):
        pltpu.async_copy(v_hbm.at[src_v.at[j]], rows, sem).wait()

        def _scale(g, c2):
            ew16 = ew_v[j, pl.ds(g * 16, 16)]
            for l in range(16):
                s = ew16[l]
                r = g * 16 + l
                for k in range(nv):
                    rows[r, pl.ds(k * 16, 16)] = rows[r, pl.ds(k * 16, 16)] * s
            return c2

        lax.fori_loop(0, CH // 16, _scale, 0)
        pltpu.sync_copy(rows, acc_sh.at[dst_v.at[j]], add=True)
        return c

    lax.fori_loop(0, NCHUNK, _chunk, 0)
    plsc.subcore_barrier()
    pltpu.sync_copy(
        acc_sh.at[pl.ds(sid * ROWS_PT, ROWS_PT)],
        out.at[cid, pl.ds(sid * ROWS_PT, ROWS_PT)],
    )


def _make_agg(d):
    return pl.kernel(
        functools.partial(_agg_body, d=d),
        out_type=jax.ShapeDtypeStruct((NC, NPAD, DW), jnp.float32),
        mesh=_MESH,
        scratch_types=[
            pltpu.VMEM((NCHUNK, CH), jnp.int32),
            pltpu.VMEM((NCHUNK, CH), jnp.int32),
            pltpu.VMEM((NCHUNK, CH), jnp.float32),
            pltpu.VMEM((CH, DW), jnp.float32),
            pltpu.VMEM_SHARED((NPAD, DW), jnp.float32),
            pltpu.SemaphoreType.DMA,
        ],
    )


_agg1_call = _make_agg(D1)
_agg2_call = _make_agg(D2)


def _mm1_body(x_ref, w1_ref, deg_ref, v1_ref, dis_ref):
    deg = deg_ref[0, :] + deg_ref[1, :] + 1.0
    dis = lax.rsqrt(deg)[:, None]
    u = jnp.dot(x_ref[...], w1_ref[...], preferred_element_type=jnp.float32)
    v1_ref[...] = u * dis
    dis_ref[...] = dis


_mm1_call = pl.pallas_call(
    _mm1_body,
    grid=(NPAD // BLK,),
    in_specs=[
        pl.BlockSpec((BLK, 128), lambda i: (i, 0)),
        pl.BlockSpec((128, DW), lambda i: (0, 0)),
        pl.BlockSpec((NC, BLK), lambda i: (0, i)),
    ],
    out_specs=[
        pl.BlockSpec((BLK, DW), lambda i: (i, 0)),
        pl.BlockSpec((BLK, 1), lambda i: (i, 0)),
    ],
    out_shape=[
        jax.ShapeDtypeStruct((NPAD, DW), jnp.float32),
        jax.ShapeDtypeStruct((NPAD, 1), jnp.float32),
    ],
)


def _mm2_body(acc_ref, v1_ref, dis_ref, gamma_ref, beta_ref, b1_ref, w2_ref, v2_ref):
    dis = dis_ref[...]
    out1 = dis * (acc_ref[0] + acc_ref[1] + v1_ref[...]) + b1_ref[...]
    h = jnp.maximum(out1 * CBN * gamma_ref[...] + beta_ref[...], 0.0)
    u2 = jnp.dot(h, w2_ref[...], preferred_element_type=jnp.float32)
    v2_ref[...] = u2 * dis


_mm2_call = pl.pallas_call(
    _mm2_body,
    grid=(NPAD // BLK,),
    in_specs=[
        pl.BlockSpec((NC, BLK, DW), lambda i: (0, i, 0)),
        pl.BlockSpec((BLK, DW), lambda i: (i, 0)),
        pl.BlockSpec((BLK, 1), lambda i: (i, 0)),
        pl.BlockSpec((1, DW), lambda i: (0, 0)),
        pl.BlockSpec((1, DW), lambda i: (0, 0)),
        pl.BlockSpec((1, DW), lambda i: (0, 0)),
        pl.BlockSpec((DW, DW), lambda i: (0, 0)),
    ],
    out_specs=pl.BlockSpec((BLK, DW), lambda i: (i, 0)),
    out_shape=jax.ShapeDtypeStruct((NPAD, DW), jnp.float32),
)


def _fin_body(acc_ref, v2_ref, dis_ref, b2_ref, out_ref):
    out_ref[...] = (
        dis_ref[...] * (acc_ref[0] + acc_ref[1] + v2_ref[...]) + b2_ref[...]
    )


_fin_call = pl.pallas_call(
    _fin_body,
    grid=(NPAD // BLK,),
    in_specs=[
        pl.BlockSpec((NC, BLK, DW), lambda i: (0, i, 0)),
        pl.BlockSpec((BLK, DW), lambda i: (i, 0)),
        pl.BlockSpec((BLK, 1), lambda i: (i, 0)),
        pl.BlockSpec((1, DW), lambda i: (0, 0)),
    ],
    out_specs=pl.BlockSpec((BLK, DW), lambda i: (i, 0)),
    out_shape=jax.ShapeDtypeStruct((NPAD, DW), jnp.float32),
)


def kernel(x, edge_index, edge_attr, W1, b1, gamma, beta, W2, b2):
    xp = jnp.zeros((NPAD, 128), jnp.float32).at[:N].set(x)
    pad = EPAD - E
    spread = jnp.arange(pad, dtype=jnp.int32) % N
    srcp = jnp.concatenate([edge_index[0], spread]).reshape(NW, NCHUNK, CH)
    dstp = jnp.concatenate([edge_index[1], spread]).reshape(NW, NCHUNK, CH)
    ewp = jnp.concatenate(
        [edge_attr, jnp.zeros((pad,), jnp.float32)]
    ).reshape(NW, NCHUNK, CH)
    w1p = jnp.zeros((128, DW), jnp.float32).at[:, :D1].set(W1)
    w2p = jnp.zeros((DW, DW), jnp.float32).at[:D1, : W2.shape[1]].set(W2)
    b1p = jnp.zeros((1, DW), jnp.float32).at[0, :D1].set(b1)
    gp = jnp.zeros((1, DW), jnp.float32).at[0, :D1].set(gamma)
    bp = jnp.zeros((1, DW), jnp.float32).at[0, :D1].set(beta)
    b2p = jnp.zeros((1, DW), jnp.float32).at[0, : b2.shape[0]].set(b2)

    deg = _deg_call(dstp, ewp)
    v1, dis = _mm1_call(xp, w1p, deg)
    acc1 = _agg1_call(v1, srcp, dstp, ewp)
    v2 = _mm2_call(acc1, v1, dis, gp, bp, b1p, w2p)
    acc2 = _agg2_call(v2, srcp, dstp, ewp)
    outp = _fin_call(acc2, v2, dis, b2p)
    return outp[:N, : b2.shape[0]]


# speedup vs baseline: 30.9029x; 1.5263x over previous
import functools
import math

import jax
import jax.numpy as jnp
from jax import lax
from jax.experimental import pallas as pl
from jax.experimental.pallas import tpu as pltpu
from jax.experimental.pallas import tpu_sc as plsc

N = 10000
NPAD = 10240
E = 320000
NC, NS = 2, 16
NW = NC * NS
CH = 128
NCHUNK = 80
EPAD = NW * NCHUNK * CH
D1 = 32
D2 = 48
DW = 128
ROWS_PT = NPAD // NS
CBN = 1.0 / math.sqrt(1.0 + 1e-5)
BLK = 1024

_MESH = plsc.VectorSubcoreMesh(
    core_axis_name="c", subcore_axis_name="s", num_cores=NC, num_subcores=NS
)


def _deg_body(dsts, ews, out, dst_v, ew_v, deg_sh):
    cid = lax.axis_index("c")
    sid = lax.axis_index("s")
    w = sid * NC + cid

    def _z(i, c):
        ew_v[0, pl.ds(i * 16, 16)] = jnp.zeros((16,), jnp.float32)
        return c

    lax.fori_loop(0, CH // 16, _z, 0)
    for t in range(ROWS_PT // CH):
        pltpu.sync_copy(ew_v.at[0], deg_sh.at[pl.ds(sid * ROWS_PT + t * CH, CH)])
    plsc.subcore_barrier()

    pltpu.sync_copy(dsts.at[w], dst_v)
    pltpu.sync_copy(ews.at[w], ew_v)

    def _chunk(j, c):
        pltpu.sync_copy(ew_v.at[j], deg_sh.at[dst_v.at[j]], add=True)
        return c

    lax.fori_loop(0, NCHUNK, _chunk, 0)
    plsc.subcore_barrier()
    pltpu.sync_copy(
        deg_sh.at[pl.ds(sid * ROWS_PT, ROWS_PT)],
        out.at[cid, pl.ds(sid * ROWS_PT, ROWS_PT)],
    )


_deg_call = pl.kernel(
    _deg_body,
    out_type=jax.ShapeDtypeStruct((NC, NPAD), jnp.float32),
    mesh=_MESH,
    scratch_types=[
        pltpu.VMEM((NCHUNK, CH), jnp.int32),
        pltpu.VMEM((NCHUNK, CH), jnp.float32),
        pltpu.VMEM_SHARED((NPAD,), jnp.float32),
    ],
)


def _agg_body(v_hbm, srcs, dsts, ews, out, src_v, dstm0, dstm1, ewm0, ewm1,
              rows0, rows1, acc_sh, gs0, gs1, ds0, ds1, es0, es1, *, d):
    cid = lax.axis_index("c")
    sid = lax.axis_index("s")
    w = sid * NC + cid
    nv = d // 16

    def _z(i, c):
        for k in range(DW // 16):
            rows0[i, pl.ds(k * 16, 16)] = jnp.zeros((16,), jnp.float32)
        return c

    lax.fori_loop(0, CH, _z, 0)
    for t in range(ROWS_PT // CH):
        pltpu.sync_copy(rows0, acc_sh.at[pl.ds(sid * ROWS_PT + t * CH, CH)])
    plsc.subcore_barrier()

    pltpu.sync_copy(srcs.at[w], src_v)

    def _process(rows, dstm, ewm):
        def _scale(g, c2):
            ew16 = ewm[0, pl.ds(g * 16, 16)]
            for l in range(16):
                s = ew16[l]
                r = g * 16 + l
                for k in range(nv):
                    rows[r, pl.ds(k * 16, 16)] = rows[r, pl.ds(k * 16, 16)] * s
            return c2

        lax.fori_loop(0, CH // 16, _scale, 0)
        pltpu.sync_copy(rows, acc_sh.at[dstm.at[0]], add=True)

    pltpu.async_copy(v_hbm.at[src_v.at[0]], rows0, gs0)
    pltpu.async_copy(dsts.at[w, 0], dstm0, ds0)
    pltpu.async_copy(ews.at[w, 0], ewm0, es0)

    def _step(j, rows, dstm, ewm, gs, dsem, esem,
              nrows, ndstm, newm, ngs, ndsem, nesem):
        @pl.when(j + 1 < NCHUNK)
        def _():
            pltpu.async_copy(v_hbm.at[src_v.at[j + 1]], nrows, ngs)
            pltpu.async_copy(dsts.at[w, j + 1], ndstm, ndsem)
            pltpu.async_copy(ews.at[w, j + 1], newm, nesem)

        pltpu.make_async_copy(v_hbm.at[src_v.at[j]], rows, gs).wait()
        pltpu.make_async_copy(dsts.at[w, j], dstm, dsem).wait()
        pltpu.make_async_copy(ews.at[w, j], ewm, esem).wait()
        _process(rows, dstm, ewm)

    def _pair(i, c):
        j0 = 2 * i
        _step(j0, rows0, dstm0, ewm0, gs0, ds0, es0,
              rows1, dstm1, ewm1, gs1, ds1, es1)
        _step(j0 + 1, rows1, dstm1, ewm1, gs1, ds1, es1,
              rows0, dstm0, ewm0, gs0, ds0, es0)
        return c

    lax.fori_loop(0, NCHUNK // 2, _pair, 0)
    plsc.subcore_barrier()
    pltpu.sync_copy(
        acc_sh.at[pl.ds(sid * ROWS_PT, ROWS_PT)],
        out.at[cid, pl.ds(sid * ROWS_PT, ROWS_PT)],
    )


def _make_agg(d):
    return pl.kernel(
        functools.partial(_agg_body, d=d),
        out_type=jax.ShapeDtypeStruct((NC, NPAD, DW), jnp.float32),
        mesh=_MESH,
        scratch_types=[
            pltpu.VMEM((NCHUNK, CH), jnp.int32),
            pltpu.VMEM((1, CH), jnp.int32),
            pltpu.VMEM((1, CH), jnp.int32),
            pltpu.VMEM((1, CH), jnp.float32),
            pltpu.VMEM((1, CH), jnp.float32),
            pltpu.VMEM((CH, DW), jnp.float32),
            pltpu.VMEM((CH, DW), jnp.float32),
            pltpu.VMEM_SHARED((NPAD, DW), jnp.float32),
            pltpu.SemaphoreType.DMA,
            pltpu.SemaphoreType.DMA,
            pltpu.SemaphoreType.DMA,
            pltpu.SemaphoreType.DMA,
            pltpu.SemaphoreType.DMA,
            pltpu.SemaphoreType.DMA,
        ],
    )


_agg1_call = _make_agg(D1)
_agg2_call = _make_agg(D2)


def _mm1_body(x_ref, w1_ref, deg_ref, v1_ref, dis_ref):
    deg = deg_ref[0, :] + deg_ref[1, :] + 1.0
    dis = lax.rsqrt(deg)[:, None]
    u = jnp.dot(x_ref[...], w1_ref[...], preferred_element_type=jnp.float32)
    v1_ref[...] = u * dis
    dis_ref[...] = dis


_mm1_call = pl.pallas_call(
    _mm1_body,
    grid=(NPAD // BLK,),
    in_specs=[
        pl.BlockSpec((BLK, 128), lambda i: (i, 0)),
        pl.BlockSpec((128, DW), lambda i: (0, 0)),
        pl.BlockSpec((NC, BLK), lambda i: (0, i)),
    ],
    out_specs=[
        pl.BlockSpec((BLK, DW), lambda i: (i, 0)),
        pl.BlockSpec((BLK, 1), lambda i: (i, 0)),
    ],
    out_shape=[
        jax.ShapeDtypeStruct((NPAD, DW), jnp.float32),
        jax.ShapeDtypeStruct((NPAD, 1), jnp.float32),
    ],
)


def _mm2_body(acc_ref, v1_ref, dis_ref, gamma_ref, beta_ref, b1_ref, w2_ref, v2_ref):
    dis = dis_ref[...]
    out1 = dis * (acc_ref[0] + acc_ref[1] + v1_ref[...]) + b1_ref[...]
    h = jnp.maximum(out1 * CBN * gamma_ref[...] + beta_ref[...], 0.0)
    u2 = jnp.dot(h, w2_ref[...], preferred_element_type=jnp.float32)
    v2_ref[...] = u2 * dis


_mm2_call = pl.pallas_call(
    _mm2_body,
    grid=(NPAD // BLK,),
    in_specs=[
        pl.BlockSpec((NC, BLK, DW), lambda i: (0, i, 0)),
        pl.BlockSpec((BLK, DW), lambda i: (i, 0)),
        pl.BlockSpec((BLK, 1), lambda i: (i, 0)),
        pl.BlockSpec((1, DW), lambda i: (0, 0)),
        pl.BlockSpec((1, DW), lambda i: (0, 0)),
        pl.BlockSpec((1, DW), lambda i: (0, 0)),
        pl.BlockSpec((DW, DW), lambda i: (0, 0)),
    ],
    out_specs=pl.BlockSpec((BLK, DW), lambda i: (i, 0)),
    out_shape=jax.ShapeDtypeStruct((NPAD, DW), jnp.float32),
)


def _fin_body(acc_ref, v2_ref, dis_ref, b2_ref, out_ref):
    out_ref[...] = (
        dis_ref[...] * (acc_ref[0] + acc_ref[1] + v2_ref[...]) + b2_ref[...]
    )


_fin_call = pl.pallas_call(
    _fin_body,
    grid=(NPAD // BLK,),
    in_specs=[
        pl.BlockSpec((NC, BLK, DW), lambda i: (0, i, 0)),
        pl.BlockSpec((BLK, DW), lambda i: (i, 0)),
        pl.BlockSpec((BLK, 1), lambda i: (i, 0)),
        pl.BlockSpec((1, DW), lambda i: (0, 0)),
    ],
    out_specs=pl.BlockSpec((BLK, DW), lambda i: (i, 0)),
    out_shape=jax.ShapeDtypeStruct((NPAD, DW), jnp.float32),
)


def kernel(x, edge_index, edge_attr, W1, b1, gamma, beta, W2, b2):
    xp = jnp.zeros((NPAD, 128), jnp.float32).at[:N].set(x)
    pad = EPAD - E
    spread = jnp.arange(pad, dtype=jnp.int32) % N
    srcp = jnp.concatenate([edge_index[0], spread]).reshape(NW, NCHUNK, CH)
    dstp = jnp.concatenate([edge_index[1], spread]).reshape(NW, NCHUNK, CH)
    ewp = jnp.concatenate(
        [edge_attr, jnp.zeros((pad,), jnp.float32)]
    ).reshape(NW, NCHUNK, CH)
    w1p = jnp.zeros((128, DW), jnp.float32).at[:, :D1].set(W1)
    w2p = jnp.zeros((DW, DW), jnp.float32).at[:D1, : W2.shape[1]].set(W2)
    b1p = jnp.zeros((1, DW), jnp.float32).at[0, :D1].set(b1)
    gp = jnp.zeros((1, DW), jnp.float32).at[0, :D1].set(gamma)
    bp = jnp.zeros((1, DW), jnp.float32).at[0, :D1].set(beta)
    b2p = jnp.zeros((1, DW), jnp.float32).at[0, : b2.shape[0]].set(b2)

    deg = _deg_call(dstp, ewp)
    v1, dis = _mm1_call(xp, w1p, deg)
    dstp_r = dstp.reshape(NW, NCHUNK, 1, CH)
    ewp_r = ewp.reshape(NW, NCHUNK, 1, CH)
    acc1 = _agg1_call(v1, srcp, dstp_r, ewp_r)
    v2 = _mm2_call(acc1, v1, dis, gp, bp, b1p, w2p)
    acc2 = _agg2_call(v2, srcp, dstp_r, ewp_r)
    outp = _fin_call(acc2, v2, dis, b2p)
    return outp[:N, : b2.shape[0]]


# speedup vs baseline: 31.5291x; 1.0203x over previous
import functools
import math

import jax
import jax.numpy as jnp
from jax import lax
from jax.experimental import pallas as pl
from jax.experimental.pallas import tpu as pltpu
from jax.experimental.pallas import tpu_sc as plsc

N = 10000
NPAD = 10240
E = 320000
NC, NS = 2, 16
NW = NC * NS
CH = 64
NCHUNK = 162
EPAD = NW * NCHUNK * CH
D1 = 32
D2 = 48
DW = 128
ROWS_PT = NPAD // NS
CBN = 1.0 / math.sqrt(1.0 + 1e-5)
BLK = 1024

_MESH = plsc.VectorSubcoreMesh(
    core_axis_name="c", subcore_axis_name="s", num_cores=NC, num_subcores=NS
)


def _deg_body(dsts, ews, out, dst_v, ew_v, deg_sh):
    cid = lax.axis_index("c")
    sid = lax.axis_index("s")
    w = sid * NC + cid

    def _z(i, c):
        ew_v[0, pl.ds(i * 16, 16)] = jnp.zeros((16,), jnp.float32)
        return c

    lax.fori_loop(0, CH // 16, _z, 0)
    for t in range(ROWS_PT // CH):
        pltpu.sync_copy(ew_v.at[0], deg_sh.at[pl.ds(sid * ROWS_PT + t * CH, CH)])
    plsc.subcore_barrier()

    pltpu.sync_copy(dsts.at[w], dst_v)
    pltpu.sync_copy(ews.at[w], ew_v)

    def _chunk(j, c):
        pltpu.sync_copy(ew_v.at[j], deg_sh.at[dst_v.at[j]], add=True)
        return c

    lax.fori_loop(0, NCHUNK, _chunk, 0)
    plsc.subcore_barrier()
    pltpu.sync_copy(
        deg_sh.at[pl.ds(sid * ROWS_PT, ROWS_PT)],
        out.at[cid, pl.ds(sid * ROWS_PT, ROWS_PT)],
    )


_deg_call = pl.kernel(
    _deg_body,
    out_type=jax.ShapeDtypeStruct((NC, NPAD), jnp.float32),
    mesh=_MESH,
    scratch_types=[
        pltpu.VMEM((NCHUNK, CH), jnp.int32),
        pltpu.VMEM((NCHUNK, CH), jnp.float32),
        pltpu.VMEM_SHARED((NPAD,), jnp.float32),
    ],
)


def _agg_body(v_hbm, srcs, dsts, ews, out, src_v,
              dstm0, dstm1, dstm2, ewm0, ewm1, ewm2,
              rows0, rows1, rows2, acc_sh,
              gs0, gs1, gs2, ds0, ds1, ds2, es0, es1, es2, *, d):
    cid = lax.axis_index("c")
    sid = lax.axis_index("s")
    w = sid * NC + cid
    nv = d // 16

    def _z(i, c):
        for k in range(DW // 16):
            rows0[i, pl.ds(k * 16, 16)] = jnp.zeros((16,), jnp.float32)
        return c

    lax.fori_loop(0, CH, _z, 0)
    for t in range(ROWS_PT // CH):
        pltpu.sync_copy(rows0, acc_sh.at[pl.ds(sid * ROWS_PT + t * CH, CH)])
    plsc.subcore_barrier()

    pltpu.sync_copy(srcs.at[w], src_v)

    def _process(rows, dstm, ewm):
        def _scale(g, c2):
            ew16 = ewm[0, pl.ds(g * 16, 16)]
            for l in range(16):
                s = ew16[l]
                r = g * 16 + l
                for k in range(nv):
                    rows[r, pl.ds(k * 16, 16)] = rows[r, pl.ds(k * 16, 16)] * s
            return c2

        lax.fori_loop(0, CH // 16, _scale, 0)
        pltpu.sync_copy(rows, acc_sh.at[dstm.at[0]], add=True)

    slots = [
        (rows0, dstm0, ewm0, gs0, ds0, es0),
        (rows1, dstm1, ewm1, gs1, ds1, es1),
        (rows2, dstm2, ewm2, gs2, ds2, es2),
    ]

    def _prefetch(j, slot):
        rows, dstm, ewm, gs, dsem, esem = slot
        pltpu.async_copy(v_hbm.at[src_v.at[j]], rows, gs)
        pltpu.async_copy(dsts.at[w, j], dstm, dsem)
        pltpu.async_copy(ews.at[w, j], ewm, esem)

    for p in range(2):
        _prefetch(p, slots[p])

    def _step(j, slot, nslot):
        @pl.when(j + 2 < NCHUNK)
        def _():
            _prefetch(j + 2, nslot)

        rows, dstm, ewm, gs, dsem, esem = slot
        pltpu.make_async_copy(v_hbm.at[src_v.at[j]], rows, gs).wait()
        pltpu.make_async_copy(dsts.at[w, j], dstm, dsem).wait()
        pltpu.make_async_copy(ews.at[w, j], ewm, esem).wait()
        _process(rows, dstm, ewm)

    def _tri(i, c):
        j0 = 3 * i
        for q in range(3):
            _step(j0 + q, slots[q], slots[(q + 2) % 3])
        return c

    lax.fori_loop(0, NCHUNK // 3, _tri, 0)
    plsc.subcore_barrier()
    pltpu.sync_copy(
        acc_sh.at[pl.ds(sid * ROWS_PT, ROWS_PT)],
        out.at[cid, pl.ds(sid * ROWS_PT, ROWS_PT)],
    )


def _make_agg(d):
    return pl.kernel(
        functools.partial(_agg_body, d=d),
        out_type=jax.ShapeDtypeStruct((NC, NPAD, DW), jnp.float32),
        mesh=_MESH,
        scratch_types=(
            [pltpu.VMEM((NCHUNK, CH), jnp.int32)]
            + [pltpu.VMEM((1, CH), jnp.int32) for _ in range(3)]
            + [pltpu.VMEM((1, CH), jnp.float32) for _ in range(3)]
            + [pltpu.VMEM((CH, DW), jnp.float32) for _ in range(3)]
            + [pltpu.VMEM_SHARED((NPAD, DW), jnp.float32)]
            + [pltpu.SemaphoreType.DMA for _ in range(9)]
        ),
    )


_agg1_call = _make_agg(D1)
_agg2_call = _make_agg(D2)


def _mm1_body(x_ref, w1_ref, deg_ref, v1_ref, dis_ref):
    deg = deg_ref[0, :] + deg_ref[1, :] + 1.0
    dis = lax.rsqrt(deg)[:, None]
    u = jnp.dot(x_ref[...], w1_ref[...], preferred_element_type=jnp.float32)
    v1_ref[...] = u * dis
    dis_ref[...] = dis


_mm1_call = pl.pallas_call(
    _mm1_body,
    grid=(NPAD // BLK,),
    in_specs=[
        pl.BlockSpec((BLK, 128), lambda i: (i, 0)),
        pl.BlockSpec((128, DW), lambda i: (0, 0)),
        pl.BlockSpec((NC, BLK), lambda i: (0, i)),
    ],
    out_specs=[
        pl.BlockSpec((BLK, DW), lambda i: (i, 0)),
        pl.BlockSpec((BLK, 1), lambda i: (i, 0)),
    ],
    out_shape=[
        jax.ShapeDtypeStruct((NPAD, DW), jnp.float32),
        jax.ShapeDtypeStruct((NPAD, 1), jnp.float32),
    ],
)


def _mm2_body(acc_ref, v1_ref, dis_ref, gamma_ref, beta_ref, b1_ref, w2_ref, v2_ref):
    dis = dis_ref[...]
    out1 = dis * (acc_ref[0] + acc_ref[1] + v1_ref[...]) + b1_ref[...]
    h = jnp.maximum(out1 * CBN * gamma_ref[...] + beta_ref[...], 0.0)
    u2 = jnp.dot(h, w2_ref[...], preferred_element_type=jnp.float32)
    v2_ref[...] = u2 * dis


_mm2_call = pl.pallas_call(
    _mm2_body,
    grid=(NPAD // BLK,),
    in_specs=[
        pl.BlockSpec((NC, BLK, DW), lambda i: (0, i, 0)),
        pl.BlockSpec((BLK, DW), lambda i: (i, 0)),
        pl.BlockSpec((BLK, 1), lambda i: (i, 0)),
        pl.BlockSpec((1, DW), lambda i: (0, 0)),
        pl.BlockSpec((1, DW), lambda i: (0, 0)),
        pl.BlockSpec((1, DW), lambda i: (0, 0)),
        pl.BlockSpec((DW, DW), lambda i: (0, 0)),
    ],
    out_specs=pl.BlockSpec((BLK, DW), lambda i: (i, 0)),
    out_shape=jax.ShapeDtypeStruct((NPAD, DW), jnp.float32),
)


def _fin_body(acc_ref, v2_ref, dis_ref, b2_ref, out_ref):
    out_ref[...] = (
        dis_ref[...] * (acc_ref[0] + acc_ref[1] + v2_ref[...]) + b2_ref[...]
    )


_fin_call = pl.pallas_call(
    _fin_body,
    grid=(NPAD // BLK,),
    in_specs=[
        pl.BlockSpec((NC, BLK, DW), lambda i: (0, i, 0)),
        pl.BlockSpec((BLK, DW), lambda i: (i, 0)),
        pl.BlockSpec((BLK, 1), lambda i: (i, 0)),
        pl.BlockSpec((1, DW), lambda i: (0, 0)),
    ],
    out_specs=pl.BlockSpec((BLK, DW), lambda i: (i, 0)),
    out_shape=jax.ShapeDtypeStruct((NPAD, DW), jnp.float32),
)


def kernel(x, edge_index, edge_attr, W1, b1, gamma, beta, W2, b2):
    xp = jnp.zeros((NPAD, 128), jnp.float32).at[:N].set(x)
    pad = EPAD - E
    spread = jnp.arange(pad, dtype=jnp.int32) % N
    srcp = jnp.concatenate([edge_index[0], spread]).reshape(NW, NCHUNK, CH)
    dstp = jnp.concatenate([edge_index[1], spread]).reshape(NW, NCHUNK, CH)
    ewp = jnp.concatenate(
        [edge_attr, jnp.zeros((pad,), jnp.float32)]
    ).reshape(NW, NCHUNK, CH)
    w1p = jnp.zeros((128, DW), jnp.float32).at[:, :D1].set(W1)
    w2p = jnp.zeros((DW, DW), jnp.float32).at[:D1, : W2.shape[1]].set(W2)
    b1p = jnp.zeros((1, DW), jnp.float32).at[0, :D1].set(b1)
    gp = jnp.zeros((1, DW), jnp.float32).at[0, :D1].set(gamma)
    bp = jnp.zeros((1, DW), jnp.float32).at[0, :D1].set(beta)
    b2p = jnp.zeros((1, DW), jnp.float32).at[0, : b2.shape[0]].set(b2)

    deg = _deg_call(dstp, ewp)
    v1, dis = _mm1_call(xp, w1p, deg)
    dstp_r = dstp.reshape(NW, NCHUNK, 1, CH)
    ewp_r = ewp.reshape(NW, NCHUNK, 1, CH)
    acc1 = _agg1_call(v1, srcp, dstp_r, ewp_r)
    v2 = _mm2_call(acc1, v1, dis, gp, bp, b1p, w2p)
    acc2 = _agg2_call(v2, srcp, dstp_r, ewp_r)
    outp = _fin_call(acc2, v2, dis, b2p)
    return outp[:N, : b2.shape[0]]


# speedup vs baseline: 31.6795x; 1.0048x over previous
import functools
import math

import jax
import jax.numpy as jnp
from jax import lax
from jax.experimental import pallas as pl
from jax.experimental.pallas import tpu as pltpu
from jax.experimental.pallas import tpu_sc as plsc

N = 10000
NPAD = 10240
E = 320000
NC, NS = 2, 16
NW = NC * NS
CH = 80
NCHUNK = 129
EPAD = NW * NCHUNK * CH
D1 = 32
D2 = 48
DW = 128
ROWS_PT = NPAD // NS
CBN = 1.0 / math.sqrt(1.0 + 1e-5)
BLK = 1024

_MESH = plsc.VectorSubcoreMesh(
    core_axis_name="c", subcore_axis_name="s", num_cores=NC, num_subcores=NS
)


def _deg_body(dsts, ews, out, dst_v, ew_v, deg_sh):
    cid = lax.axis_index("c")
    sid = lax.axis_index("s")
    w = sid * NC + cid

    def _z(i, c):
        ew_v[0, pl.ds(i * 16, 16)] = jnp.zeros((16,), jnp.float32)
        return c

    lax.fori_loop(0, CH // 16, _z, 0)
    for t in range(ROWS_PT // CH):
        pltpu.sync_copy(ew_v.at[0], deg_sh.at[pl.ds(sid * ROWS_PT + t * CH, CH)])
    plsc.subcore_barrier()

    pltpu.sync_copy(dsts.at[w], dst_v)
    pltpu.sync_copy(ews.at[w], ew_v)

    def _chunk(j, c):
        pltpu.sync_copy(ew_v.at[j], deg_sh.at[dst_v.at[j]], add=True)
        return c

    lax.fori_loop(0, NCHUNK, _chunk, 0)
    plsc.subcore_barrier()
    pltpu.sync_copy(
        deg_sh.at[pl.ds(sid * ROWS_PT, ROWS_PT)],
        out.at[cid, pl.ds(sid * ROWS_PT, ROWS_PT)],
    )


_deg_call = pl.kernel(
    _deg_body,
    out_type=jax.ShapeDtypeStruct((NC, NPAD), jnp.float32),
    mesh=_MESH,
    scratch_types=[
        pltpu.VMEM((NCHUNK, CH), jnp.int32),
        pltpu.VMEM((NCHUNK, CH), jnp.float32),
        pltpu.VMEM_SHARED((NPAD,), jnp.float32),
    ],
)


def _agg_body(v_hbm, srcs, dsts, ews, out, src_v,
              dstm0, dstm1, dstm2, ewm0, ewm1, ewm2,
              rows0, rows1, rows2, acc_sh,
              gs0, gs1, gs2, ds0, ds1, ds2, es0, es1, es2, *, d):
    cid = lax.axis_index("c")
    sid = lax.axis_index("s")
    w = sid * NC + cid
    nv = d // 16

    def _z(i, c):
        for k in range(DW // 16):
            rows0[i, pl.ds(k * 16, 16)] = jnp.zeros((16,), jnp.float32)
        return c

    lax.fori_loop(0, CH, _z, 0)
    for t in range(ROWS_PT // CH):
        pltpu.sync_copy(rows0, acc_sh.at[pl.ds(sid * ROWS_PT + t * CH, CH)])
    plsc.subcore_barrier()

    pltpu.sync_copy(srcs.at[w], src_v)

    def _process(rows, dstm, ewm):
        def _scale(g, c2):
            ew16 = ewm[0, pl.ds(g * 16, 16)]
            for l in range(16):
                s = ew16[l]
                r = g * 16 + l
                for k in range(nv):
                    rows[r, pl.ds(k * 16, 16)] = rows[r, pl.ds(k * 16, 16)] * s
            return c2

        lax.fori_loop(0, CH // 16, _scale, 0)
        pltpu.sync_copy(rows, acc_sh.at[dstm.at[0]], add=True)

    slots = [
        (rows0, dstm0, ewm0, gs0, ds0, es0),
        (rows1, dstm1, ewm1, gs1, ds1, es1),
        (rows2, dstm2, ewm2, gs2, ds2, es2),
    ]

    def _prefetch(j, slot):
        rows, dstm, ewm, gs, dsem, esem = slot
        pltpu.async_copy(v_hbm.at[src_v.at[j]], rows, gs)
        pltpu.async_copy(dsts.at[w, j], dstm, dsem)
        pltpu.async_copy(ews.at[w, j], ewm, esem)

    for p in range(2):
        _prefetch(p, slots[p])

    def _step(j, slot, nslot):
        @pl.when(j + 2 < NCHUNK)
        def _():
            _prefetch(j + 2, nslot)

        rows, dstm, ewm, gs, dsem, esem = slot
        pltpu.make_async_copy(v_hbm.at[src_v.at[j]], rows, gs).wait()
        pltpu.make_async_copy(dsts.at[w, j], dstm, dsem).wait()
        pltpu.make_async_copy(ews.at[w, j], ewm, esem).wait()
        _process(rows, dstm, ewm)

    def _tri(i, c):
        j0 = 3 * i
        for q in range(3):
            _step(j0 + q, slots[q], slots[(q + 2) % 3])
        return c

    lax.fori_loop(0, NCHUNK // 3, _tri, 0)
    plsc.subcore_barrier()
    pltpu.sync_copy(
        acc_sh.at[pl.ds(sid * ROWS_PT, ROWS_PT)],
        out.at[cid, pl.ds(sid * ROWS_PT, ROWS_PT)],
    )


def _make_agg(d):
    return pl.kernel(
        functools.partial(_agg_body, d=d),
        out_type=jax.ShapeDtypeStruct((NC, NPAD, DW), jnp.float32),
        mesh=_MESH,
        scratch_types=(
            [pltpu.VMEM((NCHUNK, CH), jnp.int32)]
            + [pltpu.VMEM((1, CH), jnp.int32) for _ in range(3)]
            + [pltpu.VMEM((1, CH), jnp.float32) for _ in range(3)]
            + [pltpu.VMEM((CH, DW), jnp.float32) for _ in range(3)]
            + [pltpu.VMEM_SHARED((NPAD, DW), jnp.float32)]
            + [pltpu.SemaphoreType.DMA for _ in range(9)]
        ),
    )


_agg1_call = _make_agg(D1)
_agg2_call = _make_agg(D2)


def _mm1_body(x_ref, w1_ref, deg_ref, v1_ref, dis_ref):
    deg = deg_ref[0, :] + deg_ref[1, :] + 1.0
    dis = lax.rsqrt(deg)[:, None]
    u = jnp.dot(x_ref[...], w1_ref[...], preferred_element_type=jnp.float32)
    v1_ref[...] = u * dis
    dis_ref[...] = dis


_mm1_call = pl.pallas_call(
    _mm1_body,
    grid=(NPAD // BLK,),
    in_specs=[
        pl.BlockSpec((BLK, 128), lambda i: (i, 0)),
        pl.BlockSpec((128, DW), lambda i: (0, 0)),
        pl.BlockSpec((NC, BLK), lambda i: (0, i)),
    ],
    out_specs=[
        pl.BlockSpec((BLK, DW), lambda i: (i, 0)),
        pl.BlockSpec((BLK, 1), lambda i: (i, 0)),
    ],
    out_shape=[
        jax.ShapeDtypeStruct((NPAD, DW), jnp.float32),
        jax.ShapeDtypeStruct((NPAD, 1), jnp.float32),
    ],
)


def _mm2_body(acc_ref, v1_ref, dis_ref, gamma_ref, beta_ref, b1_ref, w2_ref, v2_ref):
    dis = dis_ref[...]
    out1 = dis * (acc_ref[0] + acc_ref[1] + v1_ref[...]) + b1_ref[...]
    h = jnp.maximum(out1 * CBN * gamma_ref[...] + beta_ref[...], 0.0)
    u2 = jnp.dot(h, w2_ref[...], preferred_element_type=jnp.float32)
    v2_ref[...] = u2 * dis


_mm2_call = pl.pallas_call(
    _mm2_body,
    grid=(NPAD // BLK,),
    in_specs=[
        pl.BlockSpec((NC, BLK, DW), lambda i: (0, i, 0)),
        pl.BlockSpec((BLK, DW), lambda i: (i, 0)),
        pl.BlockSpec((BLK, 1), lambda i: (i, 0)),
        pl.BlockSpec((1, DW), lambda i: (0, 0)),
        pl.BlockSpec((1, DW), lambda i: (0, 0)),
        pl.BlockSpec((1, DW), lambda i: (0, 0)),
        pl.BlockSpec((DW, DW), lambda i: (0, 0)),
    ],
    out_specs=pl.BlockSpec((BLK, DW), lambda i: (i, 0)),
    out_shape=jax.ShapeDtypeStruct((NPAD, DW), jnp.float32),
)


def _fin_body(acc_ref, v2_ref, dis_ref, b2_ref, out_ref):
    out_ref[...] = (
        dis_ref[...] * (acc_ref[0] + acc_ref[1] + v2_ref[...]) + b2_ref[...]
    )


_fin_call = pl.pallas_call(
    _fin_body,
    grid=(NPAD // BLK,),
    in_specs=[
        pl.BlockSpec((NC, BLK, DW), lambda i: (0, i, 0)),
        pl.BlockSpec((BLK, DW), lambda i: (i, 0)),
        pl.BlockSpec((BLK, 1), lambda i: (i, 0)),
        pl.BlockSpec((1, DW), lambda i: (0, 0)),
    ],
    out_specs=pl.BlockSpec((BLK, DW), lambda i: (i, 0)),
    out_shape=jax.ShapeDtypeStruct((NPAD, DW), jnp.float32),
)


def kernel(x, edge_index, edge_attr, W1, b1, gamma, beta, W2, b2):
    xp = jnp.zeros((NPAD, 128), jnp.float32).at[:N].set(x)
    pad = EPAD - E
    spread = jnp.arange(pad, dtype=jnp.int32) % N
    srcp = jnp.concatenate([edge_index[0], spread]).reshape(NW, NCHUNK, CH)
    dstp = jnp.concatenate([edge_index[1], spread]).reshape(NW, NCHUNK, CH)
    ewp = jnp.concatenate(
        [edge_attr, jnp.zeros((pad,), jnp.float32)]
    ).reshape(NW, NCHUNK, CH)
    w1p = jnp.zeros((128, DW), jnp.float32).at[:, :D1].set(W1)
    w2p = jnp.zeros((DW, DW), jnp.float32).at[:D1, : W2.shape[1]].set(W2)
    b1p = jnp.zeros((1, DW), jnp.float32).at[0, :D1].set(b1)
    gp = jnp.zeros((1, DW), jnp.float32).at[0, :D1].set(gamma)
    bp = jnp.zeros((1, DW), jnp.float32).at[0, :D1].set(beta)
    b2p = jnp.zeros((1, DW), jnp.float32).at[0, : b2.shape[0]].set(b2)

    deg = _deg_call(dstp, ewp)
    v1, dis = _mm1_call(xp, w1p, deg)
    dstp_r = dstp.reshape(NW, NCHUNK, 1, CH)
    ewp_r = ewp.reshape(NW, NCHUNK, 1, CH)
    acc1 = _agg1_call(v1, srcp, dstp_r, ewp_r)
    v2 = _mm2_call(acc1, v1, dis, gp, bp, b1p, w2p)
    acc2 = _agg2_call(v2, srcp, dstp_r, ewp_r)
    outp = _fin_call(acc2, v2, dis, b2p)
    return outp[:N, : b2.shape[0]]


# speedup vs baseline: 32.3491x; 1.0211x over previous
import functools
import math

import jax
import jax.numpy as jnp
from jax import lax
from jax.experimental import pallas as pl
from jax.experimental.pallas import tpu as pltpu
from jax.experimental.pallas import tpu_sc as plsc

N = 10000
NPAD = 10240
E = 320000
NC, NS = 2, 16
NW = NC * NS
CH = 80
NCHUNK = 129
EPAD = NW * NCHUNK * CH
D1 = 32
D2 = 48
DW = 128
ROWS_PT = NPAD // NS
CBN = 1.0 / math.sqrt(1.0 + 1e-5)
BLK = 2048

_MESH = plsc.VectorSubcoreMesh(
    core_axis_name="c", subcore_axis_name="s", num_cores=NC, num_subcores=NS
)


def _deg_body(dsts, ews, out, dst_v, ew_v, deg_sh):
    cid = lax.axis_index("c")
    sid = lax.axis_index("s")
    w = sid * NC + cid

    def _z(i, c):
        ew_v[0, pl.ds(i * 16, 16)] = jnp.zeros((16,), jnp.float32)
        return c

    lax.fori_loop(0, CH // 16, _z, 0)
    for t in range(ROWS_PT // CH):
        pltpu.sync_copy(ew_v.at[0], deg_sh.at[pl.ds(sid * ROWS_PT + t * CH, CH)])
    plsc.subcore_barrier()

    pltpu.sync_copy(dsts.at[w], dst_v)
    pltpu.sync_copy(ews.at[w], ew_v)

    def _chunk(j, c):
        pltpu.sync_copy(ew_v.at[j], deg_sh.at[dst_v.at[j]], add=True)
        return c

    lax.fori_loop(0, NCHUNK, _chunk, 0)
    plsc.subcore_barrier()
    pltpu.sync_copy(
        deg_sh.at[pl.ds(sid * ROWS_PT, ROWS_PT)],
        out.at[cid, pl.ds(sid * ROWS_PT, ROWS_PT)],
    )


_deg_call = pl.kernel(
    _deg_body,
    out_type=jax.ShapeDtypeStruct((NC, NPAD), jnp.float32),
    mesh=_MESH,
    scratch_types=[
        pltpu.VMEM((NCHUNK, CH), jnp.int32),
        pltpu.VMEM((NCHUNK, CH), jnp.float32),
        pltpu.VMEM_SHARED((NPAD,), jnp.float32),
    ],
)


def _agg_body(v_hbm, srcs, dsts, ews, out, src_v,
              dstm0, dstm1, dstm2, ewm0, ewm1, ewm2,
              rows0, rows1, rows2, acc_sh,
              gs0, gs1, gs2, ds0, ds1, ds2, es0, es1, es2, *, d):
    cid = lax.axis_index("c")
    sid = lax.axis_index("s")
    w = sid * NC + cid
    nv = d // 16

    def _z(i, c):
        for k in range(DW // 16):
            rows0[i, pl.ds(k * 16, 16)] = jnp.zeros((16,), jnp.float32)
        return c

    lax.fori_loop(0, CH, _z, 0)
    for t in range(ROWS_PT // CH):
        pltpu.sync_copy(rows0, acc_sh.at[pl.ds(sid * ROWS_PT + t * CH, CH)])
    plsc.subcore_barrier()

    pltpu.sync_copy(srcs.at[w], src_v)

    def _process(rows, dstm, ewm):
        def _scale(g, c2):
            ew16 = ewm[0, pl.ds(g * 16, 16)]
            for l in range(16):
                s = ew16[l]
                r = g * 16 + l
                for k in range(nv):
                    rows[r, pl.ds(k * 16, 16)] = rows[r, pl.ds(k * 16, 16)] * s
            return c2

        lax.fori_loop(0, CH // 16, _scale, 0)
        pltpu.sync_copy(rows, acc_sh.at[dstm.at[0]], add=True)

    slots = [
        (rows0, dstm0, ewm0, gs0, ds0, es0),
        (rows1, dstm1, ewm1, gs1, ds1, es1),
        (rows2, dstm2, ewm2, gs2, ds2, es2),
    ]

    def _prefetch(j, slot):
        rows, dstm, ewm, gs, dsem, esem = slot
        pltpu.async_copy(v_hbm.at[src_v.at[j]], rows, gs)
        pltpu.async_copy(dsts.at[w, j], dstm, dsem)
        pltpu.async_copy(ews.at[w, j], ewm, esem)

    for p in range(2):
        _prefetch(p, slots[p])

    def _step(j, slot, nslot):
        @pl.when(j + 2 < NCHUNK)
        def _():
            _prefetch(j + 2, nslot)

        rows, dstm, ewm, gs, dsem, esem = slot
        pltpu.make_async_copy(v_hbm.at[src_v.at[j]], rows, gs).wait()
        pltpu.make_async_copy(dsts.at[w, j], dstm, dsem).wait()
        pltpu.make_async_copy(ews.at[w, j], ewm, esem).wait()
        _process(rows, dstm, ewm)

    def _tri(i, c):
        j0 = 3 * i
        for q in range(3):
            _step(j0 + q, slots[q], slots[(q + 2) % 3])
        return c

    lax.fori_loop(0, NCHUNK // 3, _tri, 0)
    plsc.subcore_barrier()
    pltpu.sync_copy(
        acc_sh.at[pl.ds(sid * ROWS_PT, ROWS_PT)],
        out.at[cid, pl.ds(sid * ROWS_PT, ROWS_PT)],
    )


def _make_agg(d):
    return pl.kernel(
        functools.partial(_agg_body, d=d),
        out_type=jax.ShapeDtypeStruct((NC, NPAD, DW), jnp.float32),
        mesh=_MESH,
        scratch_types=(
            [pltpu.VMEM((NCHUNK, CH), jnp.int32)]
            + [pltpu.VMEM((1, CH), jnp.int32) for _ in range(3)]
            + [pltpu.VMEM((1, CH), jnp.float32) for _ in range(3)]
            + [pltpu.VMEM((CH, DW), jnp.float32) for _ in range(3)]
            + [pltpu.VMEM_SHARED((NPAD, DW), jnp.float32)]
            + [pltpu.SemaphoreType.DMA for _ in range(9)]
        ),
    )


_agg1_call = _make_agg(D1)
_agg2_call = _make_agg(D2)


def _mm1_body(x_ref, w1_ref, deg_ref, v1_ref, dis_ref):
    deg = deg_ref[0, :] + deg_ref[1, :] + 1.0
    dis = lax.rsqrt(deg)[:, None]
    u = jnp.dot(x_ref[...], w1_ref[...], preferred_element_type=jnp.float32)
    v1_ref[...] = u * dis
    dis_ref[...] = dis


_mm1_call = pl.pallas_call(
    _mm1_body,
    grid=(NPAD // BLK,),
    in_specs=[
        pl.BlockSpec((BLK, 128), lambda i: (i, 0)),
        pl.BlockSpec((128, DW), lambda i: (0, 0)),
        pl.BlockSpec((NC, BLK), lambda i: (0, i)),
    ],
    out_specs=[
        pl.BlockSpec((BLK, DW), lambda i: (i, 0)),
        pl.BlockSpec((BLK, 1), lambda i: (i, 0)),
    ],
    out_shape=[
        jax.ShapeDtypeStruct((NPAD, DW), jnp.float32),
        jax.ShapeDtypeStruct((NPAD, 1), jnp.float32),
    ],
)


def _mm2_body(acc_ref, v1_ref, dis_ref, gamma_ref, beta_ref, b1_ref, w2_ref, v2_ref):
    dis = dis_ref[...]
    out1 = dis * (acc_ref[0] + acc_ref[1] + v1_ref[...]) + b1_ref[...]
    h = jnp.maximum(out1 * CBN * gamma_ref[...] + beta_ref[...], 0.0)
    u2 = jnp.dot(h, w2_ref[...], preferred_element_type=jnp.float32)
    v2_ref[...] = u2 * dis


_mm2_call = pl.pallas_call(
    _mm2_body,
    grid=(NPAD // BLK,),
    in_specs=[
        pl.BlockSpec((NC, BLK, DW), lambda i: (0, i, 0)),
        pl.BlockSpec((BLK, DW), lambda i: (i, 0)),
        pl.BlockSpec((BLK, 1), lambda i: (i, 0)),
        pl.BlockSpec((1, DW), lambda i: (0, 0)),
        pl.BlockSpec((1, DW), lambda i: (0, 0)),
        pl.BlockSpec((1, DW), lambda i: (0, 0)),
        pl.BlockSpec((DW, DW), lambda i: (0, 0)),
    ],
    out_specs=pl.BlockSpec((BLK, DW), lambda i: (i, 0)),
    out_shape=jax.ShapeDtypeStruct((NPAD, DW), jnp.float32),
)


def _fin_body(acc_ref, v2_ref, dis_ref, b2_ref, out_ref):
    out_ref[...] = (
        dis_ref[...] * (acc_ref[0] + acc_ref[1] + v2_ref[...]) + b2_ref[...]
    )


_fin_call = pl.pallas_call(
    _fin_body,
    grid=(NPAD // BLK,),
    in_specs=[
        pl.BlockSpec((NC, BLK, DW), lambda i: (0, i, 0)),
        pl.BlockSpec((BLK, DW), lambda i: (i, 0)),
        pl.BlockSpec((BLK, 1), lambda i: (i, 0)),
        pl.BlockSpec((1, DW), lambda i: (0, 0)),
    ],
    out_specs=pl.BlockSpec((BLK, DW), lambda i: (i, 0)),
    out_shape=jax.ShapeDtypeStruct((NPAD, DW), jnp.float32),
)


def kernel(x, edge_index, edge_attr, W1, b1, gamma, beta, W2, b2):
    xp = jnp.zeros((NPAD, 128), jnp.float32).at[:N].set(x)
    pad = EPAD - E
    spread = jnp.arange(pad, dtype=jnp.int32) % N
    srcp = jnp.concatenate([edge_index[0], spread]).reshape(NW, NCHUNK, CH)
    dstp = jnp.concatenate([edge_index[1], spread]).reshape(NW, NCHUNK, CH)
    ewp = jnp.concatenate(
        [edge_attr, jnp.zeros((pad,), jnp.float32)]
    ).reshape(NW, NCHUNK, CH)
    w1p = jnp.zeros((128, DW), jnp.float32).at[:, :D1].set(W1)
    w2p = jnp.zeros((DW, DW), jnp.float32).at[:D1, : W2.shape[1]].set(W2)
    b1p = jnp.zeros((1, DW), jnp.float32).at[0, :D1].set(b1)
    gp = jnp.zeros((1, DW), jnp.float32).at[0, :D1].set(gamma)
    bp = jnp.zeros((1, DW), jnp.float32).at[0, :D1].set(beta)
    b2p = jnp.zeros((1, DW), jnp.float32).at[0, : b2.shape[0]].set(b2)

    deg = _deg_call(dstp, ewp)
    v1, dis = _mm1_call(xp, w1p, deg)
    dstp_r = dstp.reshape(NW, NCHUNK, 1, CH)
    ewp_r = ewp.reshape(NW, NCHUNK, 1, CH)
    acc1 = _agg1_call(v1, srcp, dstp_r, ewp_r)
    v2 = _mm2_call(acc1, v1, dis, gp, bp, b1p, w2p)
    acc2 = _agg2_call(v2, srcp, dstp_r, ewp_r)
    outp = _fin_call(acc2, v2, dis, b2p)
    return outp[:N, : b2.shape[0]]


# speedup vs baseline: 32.5370x; 1.0058x over previous
import functools
import math

import jax
import jax.numpy as jnp
from jax import lax
from jax.experimental import pallas as pl
from jax.experimental.pallas import tpu as pltpu
from jax.experimental.pallas import tpu_sc as plsc

N = 10000
NPAD = 10240
E = 320000
NC, NS = 2, 16
NW = NC * NS
CH = 80
NCHUNK = 129
EPAD = NW * NCHUNK * CH
D1 = 32
D2 = 48
DW = 128
ROWS_PT = NPAD // NS
CBN = 1.0 / math.sqrt(1.0 + 1e-5)
BLK = 5120

_MESH = plsc.VectorSubcoreMesh(
    core_axis_name="c", subcore_axis_name="s", num_cores=NC, num_subcores=NS
)


def _deg_body(dsts, ews, out, dst_v, ew_v, deg_sh):
    cid = lax.axis_index("c")
    sid = lax.axis_index("s")
    w = sid * NC + cid

    def _z(i, c):
        ew_v[0, pl.ds(i * 16, 16)] = jnp.zeros((16,), jnp.float32)
        return c

    lax.fori_loop(0, CH // 16, _z, 0)
    for t in range(ROWS_PT // CH):
        pltpu.sync_copy(ew_v.at[0], deg_sh.at[pl.ds(sid * ROWS_PT + t * CH, CH)])
    plsc.subcore_barrier()

    pltpu.sync_copy(dsts.at[w], dst_v)
    pltpu.sync_copy(ews.at[w], ew_v)

    def _chunk(j, c):
        pltpu.sync_copy(ew_v.at[j], deg_sh.at[dst_v.at[j]], add=True)
        return c

    lax.fori_loop(0, NCHUNK, _chunk, 0)
    plsc.subcore_barrier()
    pltpu.sync_copy(
        deg_sh.at[pl.ds(sid * ROWS_PT, ROWS_PT)],
        out.at[cid, pl.ds(sid * ROWS_PT, ROWS_PT)],
    )


_deg_call = pl.kernel(
    _deg_body,
    out_type=jax.ShapeDtypeStruct((NC, NPAD), jnp.float32),
    mesh=_MESH,
    scratch_types=[
        pltpu.VMEM((NCHUNK, CH), jnp.int32),
        pltpu.VMEM((NCHUNK, CH), jnp.float32),
        pltpu.VMEM_SHARED((NPAD,), jnp.float32),
    ],
)


def _agg_body(v_hbm, srcs, dsts, ews, out, src_v,
              dstm0, dstm1, dstm2, ewm0, ewm1, ewm2,
              rows0, rows1, rows2, acc_sh,
              gs0, gs1, gs2, ds0, ds1, ds2, es0, es1, es2, *, d):
    cid = lax.axis_index("c")
    sid = lax.axis_index("s")
    w = sid * NC + cid
    nv = d // 16

    def _z(i, c):
        for k in range(DW // 16):
            rows0[i, pl.ds(k * 16, 16)] = jnp.zeros((16,), jnp.float32)
        return c

    lax.fori_loop(0, CH, _z, 0)
    for t in range(ROWS_PT // CH):
        pltpu.sync_copy(rows0, acc_sh.at[pl.ds(sid * ROWS_PT + t * CH, CH)])
    plsc.subcore_barrier()

    pltpu.sync_copy(srcs.at[w], src_v)

    def _process(rows, dstm, ewm):
        def _scale(g, c2):
            ew16 = ewm[0, pl.ds(g * 16, 16)]
            for l in range(16):
                s = ew16[l]
                r = g * 16 + l
                for k in range(nv):
                    rows[r, pl.ds(k * 16, 16)] = rows[r, pl.ds(k * 16, 16)] * s
            return c2

        lax.fori_loop(0, CH // 16, _scale, 0)
        pltpu.sync_copy(rows, acc_sh.at[dstm.at[0]], add=True)

    slots = [
        (rows0, dstm0, ewm0, gs0, ds0, es0),
        (rows1, dstm1, ewm1, gs1, ds1, es1),
        (rows2, dstm2, ewm2, gs2, ds2, es2),
    ]

    def _prefetch(j, slot):
        rows, dstm, ewm, gs, dsem, esem = slot
        pltpu.async_copy(v_hbm.at[src_v.at[j]], rows, gs)
        pltpu.async_copy(dsts.at[w, j], dstm, dsem)
        pltpu.async_copy(ews.at[w, j], ewm, esem)

    for p in range(2):
        _prefetch(p, slots[p])

    def _step(j, slot, nslot):
        @pl.when(j + 2 < NCHUNK)
        def _():
            _prefetch(j + 2, nslot)

        rows, dstm, ewm, gs, dsem, esem = slot
        pltpu.make_async_copy(v_hbm.at[src_v.at[j]], rows, gs).wait()
        pltpu.make_async_copy(dsts.at[w, j], dstm, dsem).wait()
        pltpu.make_async_copy(ews.at[w, j], ewm, esem).wait()
        _process(rows, dstm, ewm)

    def _tri(i, c):
        j0 = 3 * i
        for q in range(3):
            _step(j0 + q, slots[q], slots[(q + 2) % 3])
        return c

    lax.fori_loop(0, NCHUNK // 3, _tri, 0)
    plsc.subcore_barrier()
    pltpu.sync_copy(
        acc_sh.at[pl.ds(sid * ROWS_PT, ROWS_PT)],
        out.at[cid, pl.ds(sid * ROWS_PT, ROWS_PT)],
    )


def _make_agg(d):
    return pl.kernel(
        functools.partial(_agg_body, d=d),
        out_type=jax.ShapeDtypeStruct((NC, NPAD, DW), jnp.float32),
        mesh=_MESH,
        scratch_types=(
            [pltpu.VMEM((NCHUNK, CH), jnp.int32)]
            + [pltpu.VMEM((1, CH), jnp.int32) for _ in range(3)]
            + [pltpu.VMEM((1, CH), jnp.float32) for _ in range(3)]
            + [pltpu.VMEM((CH, DW), jnp.float32) for _ in range(3)]
            + [pltpu.VMEM_SHARED((NPAD, DW), jnp.float32)]
            + [pltpu.SemaphoreType.DMA for _ in range(9)]
        ),
    )


_agg1_call = _make_agg(D1)
_agg2_call = _make_agg(D2)


def _mm1_body(x_ref, w1_ref, deg_ref, v1_ref, dis_ref):
    deg = deg_ref[0, :] + deg_ref[1, :] + 1.0
    dis = lax.rsqrt(deg)[:, None]
    u = jnp.dot(x_ref[...], w1_ref[...], preferred_element_type=jnp.float32)
    v1_ref[...] = u * dis
    dis_ref[...] = dis


_mm1_call = pl.pallas_call(
    _mm1_body,
    grid=(NPAD // BLK,),
    in_specs=[
        pl.BlockSpec((BLK, 128), lambda i: (i, 0)),
        pl.BlockSpec((128, DW), lambda i: (0, 0)),
        pl.BlockSpec((NC, BLK), lambda i: (0, i)),
    ],
    out_specs=[
        pl.BlockSpec((BLK, DW), lambda i: (i, 0)),
        pl.BlockSpec((BLK, 1), lambda i: (i, 0)),
    ],
    out_shape=[
        jax.ShapeDtypeStruct((NPAD, DW), jnp.float32),
        jax.ShapeDtypeStruct((NPAD, 1), jnp.float32),
    ],
)


def _mm2_body(acc_ref, v1_ref, dis_ref, gamma_ref, beta_ref, b1_ref, w2_ref, v2_ref):
    dis = dis_ref[...]
    out1 = dis * (acc_ref[0] + acc_ref[1] + v1_ref[...]) + b1_ref[...]
    h = jnp.maximum(out1 * CBN * gamma_ref[...] + beta_ref[...], 0.0)
    u2 = jnp.dot(h, w2_ref[...], preferred_element_type=jnp.float32)
    v2_ref[...] = u2 * dis


_mm2_call = pl.pallas_call(
    _mm2_body,
    grid=(NPAD // BLK,),
    in_specs=[
        pl.BlockSpec((NC, BLK, DW), lambda i: (0, i, 0)),
        pl.BlockSpec((BLK, DW), lambda i: (i, 0)),
        pl.BlockSpec((BLK, 1), lambda i: (i, 0)),
        pl.BlockSpec((1, DW), lambda i: (0, 0)),
        pl.BlockSpec((1, DW), lambda i: (0, 0)),
        pl.BlockSpec((1, DW), lambda i: (0, 0)),
        pl.BlockSpec((DW, DW), lambda i: (0, 0)),
    ],
    out_specs=pl.BlockSpec((BLK, DW), lambda i: (i, 0)),
    out_shape=jax.ShapeDtypeStruct((NPAD, DW), jnp.float32),
)


def _fin_body(acc_ref, v2_ref, dis_ref, b2_ref, out_ref):
    out_ref[...] = (
        dis_ref[...] * (acc_ref[0] + acc_ref[1] + v2_ref[...]) + b2_ref[...]
    )


_fin_call = pl.pallas_call(
    _fin_body,
    grid=(NPAD // BLK,),
    in_specs=[
        pl.BlockSpec((NC, BLK, DW), lambda i: (0, i, 0)),
        pl.BlockSpec((BLK, DW), lambda i: (i, 0)),
        pl.BlockSpec((BLK, 1), lambda i: (i, 0)),
        pl.BlockSpec((1, DW), lambda i: (0, 0)),
    ],
    out_specs=pl.BlockSpec((BLK, DW), lambda i: (i, 0)),
    out_shape=jax.ShapeDtypeStruct((NPAD, DW), jnp.float32),
)


def kernel(x, edge_index, edge_attr, W1, b1, gamma, beta, W2, b2):
    xp = jnp.zeros((NPAD, 128), jnp.float32).at[:N].set(x)
    pad = EPAD - E
    spread = jnp.arange(pad, dtype=jnp.int32) % N
    srcp = jnp.concatenate([edge_index[0], spread]).reshape(NW, NCHUNK, CH)
    dstp = jnp.concatenate([edge_index[1], spread]).reshape(NW, NCHUNK, CH)
    ewp = jnp.concatenate(
        [edge_attr, jnp.zeros((pad,), jnp.float32)]
    ).reshape(NW, NCHUNK, CH)
    w1p = jnp.zeros((128, DW), jnp.float32).at[:, :D1].set(W1)
    w2p = jnp.zeros((DW, DW), jnp.float32).at[:D1, : W2.shape[1]].set(W2)
    b1p = jnp.zeros((1, DW), jnp.float32).at[0, :D1].set(b1)
    gp = jnp.zeros((1, DW), jnp.float32).at[0, :D1].set(gamma)
    bp = jnp.zeros((1, DW), jnp.float32).at[0, :D1].set(beta)
    b2p = jnp.zeros((1, DW), jnp.float32).at[0, : b2.shape[0]].set(b2)

    deg = _deg_call(dstp, ewp)
    v1, dis = _mm1_call(xp, w1p, deg)
    dstp_r = dstp.reshape(NW, NCHUNK, 1, CH)
    ewp_r = ewp.reshape(NW, NCHUNK, 1, CH)
    acc1 = _agg1_call(v1, srcp, dstp_r, ewp_r)
    v2 = _mm2_call(acc1, v1, dis, gp, bp, b1p, w2p)
    acc2 = _agg2_call(v2, srcp, dstp_r, ewp_r)
    outp = _fin_call(acc2, v2, dis, b2p)
    return outp[:N, : b2.shape[0]]
